# baseline (speedup 1.0000x reference)
import functools

import jax
import jax.numpy as jnp
from jax import lax
from jax.experimental import pallas as pl
from jax.experimental.pallas import tpu as pltpu
from jax.experimental.pallas import tpu_sc as plsc


def _tc_project_table(embed, w0b, b0, r0b):
    c, d = embed.shape
    blk = 1000
    grid = c // blk

    def body(e_ref, w_ref, b_ref, r_ref, out_ref):
        e = e_ref[...]
        out_ref[:, :d] = (
            jnp.dot(e, w_ref[...], preferred_element_type=jnp.float32)
            + b_ref[...])
        out_ref[:, d:] = jnp.dot(e, r_ref[...],
                                 preferred_element_type=jnp.float32)

    return pl.pallas_call(
        body,
        grid=(grid,),
        in_specs=[
            pl.BlockSpec((blk, d), lambda i: (i, 0)),
            pl.BlockSpec((d, d), lambda i: (0, 0)),
            pl.BlockSpec((1, d), lambda i: (0, 0)),
            pl.BlockSpec((d, d), lambda i: (0, 0)),
        ],
        out_specs=pl.BlockSpec((blk, 2 * d), lambda i: (i, 0)),
        out_shape=jax.ShapeDtypeStruct((c, 2 * d), jnp.float32),
        compiler_params=pltpu.CompilerParams(
            dimension_semantics=("arbitrary",)),
    )(embed, w0b, b0.reshape(1, d), r0b)


def _sc_gather(table, idx):
    n = idx.shape[0]
    d = table.shape[1]
    nw = 32
    per_w = n // nw
    chunk = 128
    nbuf = 6
    nstream = per_w // chunk
    ngroup = nstream // nbuf
    rem = nstream - ngroup * nbuf
    tail = per_w - nstream * chunk
    idx3 = idx.reshape(nw, 1, per_w)
    mesh = plsc.VectorSubcoreMesh(core_axis_name="c", subcore_axis_name="s")

    @functools.partial(
        pl.kernel,
        mesh=mesh,
        out_type=jax.ShapeDtypeStruct((n, d), jnp.float32),
        scratch_types=[
            pltpu.VMEM((1, per_w), jnp.int32),
            pltpu.VMEM((nbuf, chunk, d), jnp.float32),
            pltpu.SemaphoreType.DMA,
            pltpu.SemaphoreType.DMA,
        ],
    )
    def k(table_hbm, idx_hbm, out_hbm, idx_v, rows_v, gsem, wsem):
        wid = lax.axis_index("s") * 2 + lax.axis_index("c")
        pltpu.sync_copy(idx_hbm.at[wid], idx_v)
        wbase = wid * per_w

        def group(g, carry):
            @pl.when(g > 0)
            def _():
                for b in range(nbuf):
                    pltpu.make_async_copy(table_hbm.at[pl.ds(0, chunk)],
                                          rows_v.at[b], wsem).wait()
            offs = [(g * nbuf + b) * chunk for b in range(nbuf)]
            handles = [
                pltpu.async_copy(
                    table_hbm.at[idx_v.at[0, pl.ds(offs[b], chunk)]],
                    rows_v.at[b], gsem)
                for b in range(nbuf)
            ]
            for h in handles:
                h.wait()
            for b in range(nbuf):
                pltpu.async_copy(rows_v.at[b],
                                 out_hbm.at[pl.ds(wbase + offs[b], chunk)],
                                 wsem)
            return carry

        lax.fori_loop(0, ngroup, group, 0)
        if rem:
            if ngroup > 0:
                for b in range(rem):
                    pltpu.make_async_copy(table_hbm.at[pl.ds(0, chunk)],
                                          rows_v.at[b], wsem).wait()
            roffs = [(ngroup * nbuf + b) * chunk for b in range(rem)]
            rhandles = [
                pltpu.async_copy(
                    table_hbm.at[idx_v.at[0, pl.ds(roffs[b], chunk)]],
                    rows_v.at[b], gsem)
                for b in range(rem)
            ]
            for h in rhandles:
                h.wait()
            for b in range(rem):
                pltpu.async_copy(rows_v.at[b],
                                 out_hbm.at[pl.ds(wbase + roffs[b], chunk)],
                                 wsem)
        for b in range(nbuf if ngroup > 0 else rem):
            pltpu.make_async_copy(table_hbm.at[pl.ds(0, chunk)],
                                  rows_v.at[b], wsem).wait()
        if tail:
            toff = nstream * chunk
            pltpu.async_copy(
                table_hbm.at[idx_v.at[0, pl.ds(toff, tail)]],
                rows_v.at[0, pl.ds(0, tail)], gsem).wait()
            pltpu.sync_copy(rows_v.at[0, pl.ds(0, tail)],
                            out_hbm.at[pl.ds(wbase + toff, tail)])

    return k(table, idx3)


def _tc_mlp(orig, react, p):
    ws, bs, rs, w_out, b_out = p
    n, d = orig.shape
    blk = 1280
    grid = n // blk

    bf = jnp.bfloat16
    flat = [ws[0][:d].astype(bf), rs[0][:d].astype(bf)]
    layer_has_r = []
    for i in range(1, len(ws)):
        flat += [ws[i].astype(bf), bs[i].reshape(1, -1)]
        if rs[i] is not None:
            flat.append(rs[i].astype(bf))
            layer_has_r.append(True)
        else:
            layer_has_r.append(False)
    flat += [w_out.astype(bf), b_out.reshape(1, 1)]
    n_w = len(flat)

    def body(*refs):
        orig_ref, react_ref = refs[0], refs[1]
        wv = [r[...] for r in refs[2:2 + n_w]]
        gate_ref = refs[2 + n_w]
        gmax_ref = refs[3 + n_w]
        smax_ref = refs[4 + n_w]

        bf = jnp.bfloat16
        o = orig_ref[...].astype(bf)
        it = iter(wv)
        w0a, r0a = next(it), next(it)
        z = jnp.maximum(jnp.dot(o, w0a, preferred_element_type=jnp.float32)
                        + react_ref[:, :d], 0.0)
        h = z + jnp.dot(o, r0a, preferred_element_type=jnp.float32) \
              + react_ref[:, d:]
        for has_r in layer_has_r:
            w = next(it)
            b = next(it)
            h16 = h.astype(bf)
            z = jnp.maximum(jnp.dot(h16, w,
                                    preferred_element_type=jnp.float32) + b,
                            0.0)
            res = jnp.dot(h16, next(it),
                          preferred_element_type=jnp.float32) if has_r else h
            h = z + res
        wo = next(it)
        bo = next(it)
        gate = jnp.dot(h.astype(bf), wo,
                       preferred_element_type=jnp.float32) + bo[0, 0]
        gate_ref[...] = gate

        bm = jnp.max(gate)
        prev = jnp.where(pl.program_id(0) == 0, -jnp.inf, smax_ref[0])
        m = jnp.maximum(prev, bm)
        smax_ref[0] = m
        gmax_ref[...] = jnp.full((8, 128), m, jnp.float32)

    in_specs = [
        pl.BlockSpec((blk, d), lambda i: (i, 0)),
        pl.BlockSpec((blk, 2 * d), lambda i: (i, 0)),
    ]
    for a in flat:
        in_specs.append(pl.BlockSpec(a.shape, lambda i: (0,) * a.ndim))

    gate, gmax = pl.pallas_call(
        body,
        grid=(grid,),
        in_specs=in_specs,
        out_specs=[
            pl.BlockSpec((blk, 1), lambda i: (i, 0)),
            pl.BlockSpec((8, 128), lambda i: (0, 0)),
        ],
        out_shape=[
            jax.ShapeDtypeStruct((n, 1), jnp.float32),
            jax.ShapeDtypeStruct((8, 128), jnp.float32),
        ],
        scratch_shapes=[pltpu.SMEM((1,), jnp.float32)],
        compiler_params=pltpu.CompilerParams(
            dimension_semantics=("arbitrary",)),
    )(orig, react, *flat)
    return gate, gmax


def _sc_softmax(gate2, idx2, gmax16, zeros_c):
    r = gate2.shape[0]
    c = zeros_c.shape[0]
    nt = 16
    rb = r // nt
    mesh = plsc.VectorSubcoreMesh(core_axis_name="c", subcore_axis_name="s",
                                  num_cores=1)

    @functools.partial(
        pl.kernel,
        mesh=mesh,
        out_type=jax.ShapeDtypeStruct((r, 128), jnp.float32),
        scratch_types=[
            pltpu.VMEM((rb, 128), jnp.float32),
            pltpu.VMEM((rb, 128), jnp.int32),
            pltpu.VMEM((16,), jnp.float32),
            pltpu.VMEM((128,), jnp.float32),
            pltpu.VMEM_SHARED((c,), jnp.float32),
        ],
    )
    def k(gate_hbm, idx_hbm, gmax_hbm, zc_hbm, out_hbm,
          ebuf, ibuf, gmv, drow, accs):
        t = lax.axis_index("s")
        base = t * rb
        nr = rb

        pltpu.sync_copy(gate_hbm.at[pl.ds(base, rb)], ebuf)
        pltpu.sync_copy(idx_hbm.at[pl.ds(base, rb)], ibuf)
        pltpu.sync_copy(gmax_hbm, gmv)

        @pl.when(t == 0)
        def _():
            pltpu.sync_copy(zc_hbm, accs)

        m = gmv[...]

        def row_exp(ri, carry):
            for li in range(8):
                g = ebuf[ri, pl.ds(li * 16, 16)]
                ebuf[ri, pl.ds(li * 16, 16)] = jnp.exp(g - m)
            return carry

        lax.fori_loop(0, nr, row_exp, 0)
        plsc.subcore_barrier()

        def row_scatter(ri, carry):
            pltpu.sync_copy(ebuf.at[ri], accs.at[ibuf.at[ri]], add=True)
            return carry

        lax.fori_loop(0, nr, row_scatter, 0)
        plsc.subcore_barrier()

        def row_div(ri, carry):
            pltpu.sync_copy(accs.at[ibuf.at[ri]], drow)
            for li in range(8):
                den = drow[pl.ds(li * 16, 16)] + 1e-13
                e = ebuf[ri, pl.ds(li * 16, 16)]
                ebuf[ri, pl.ds(li * 16, 16)] = e / den
            return carry

        lax.fori_loop(0, nr, row_div, 0)

        pltpu.sync_copy(ebuf, out_hbm.at[pl.ds(base, rb)])

    return k(gate2, idx2, gmax16, zeros_c)


def kernel(orig_elem_fea, reaction_elem_idx, reaction_embed, params):
    n = orig_elem_fea.shape[0]
    c = reaction_embed.shape[0]
    idx = reaction_elem_idx.astype(jnp.int32)
    rows = n // 128
    rows_pad = ((rows + 127) // 128) * 128
    pad = rows_pad - rows
    idx2 = jnp.concatenate(
        [idx.reshape(rows, 128),
         jnp.full((pad, 128), c - 1, jnp.int32)], axis=0)
    zeros_c = jnp.zeros((c,), jnp.float32)

    outs = []
    nphase = 2
    ph = n // nphase
    for p in params:
        ws, bs, rs, _, _ = p
        d = orig_elem_fea.shape[1]
        table = _tc_project_table(reaction_embed, ws[0][d:], bs[0], rs[0][d:])
        gates, gmaxes = [], []
        for k in range(nphase):
            sl = slice(k * ph, (k + 1) * ph)
            react = _sc_gather(table, idx[sl])
            g, gm = _tc_mlp(orig_elem_fea[sl], react, p)
            gates.append(g)
            gmaxes.append(gm)
        gate = jnp.concatenate(gates, axis=0)
        gmax = gmaxes[0]
        for gm in gmaxes[1:]:
            gmax = jnp.maximum(gmax, gm)
        gate2 = jnp.concatenate(
            [gate.reshape(rows, 128),
             jnp.full((pad, 128), -1e30, jnp.float32)], axis=0)
        gmax16 = gmax[0, :16]
        sm = _sc_softmax(gate2, idx2, gmax16, zeros_c)
        outs.append(sm[:rows].reshape(-1))
    if len(outs) == 1:
        return outs[0]
    return jnp.mean(jnp.stack(outs), axis=0)


# speedup vs baseline: 1.2392x; 1.2392x over previous
import functools

import jax
import jax.numpy as jnp
from jax import lax
from jax.experimental import pallas as pl
from jax.experimental.pallas import tpu as pltpu
from jax.experimental.pallas import tpu_sc as plsc


def _tc_project_table(embed, w0b, b0, r0b):
    c, d = embed.shape
    blk = 1000
    grid = c // blk

    def body(e_ref, w_ref, b_ref, r_ref, out_ref):
        e = e_ref[...]
        out_ref[:, :d] = (
            jnp.dot(e, w_ref[...], preferred_element_type=jnp.float32)
            + b_ref[...])
        out_ref[:, d:] = jnp.dot(e, r_ref[...],
                                 preferred_element_type=jnp.float32)

    return pl.pallas_call(
        body,
        grid=(grid,),
        in_specs=[
            pl.BlockSpec((blk, d), lambda i: (i, 0)),
            pl.BlockSpec((d, d), lambda i: (0, 0)),
            pl.BlockSpec((1, d), lambda i: (0, 0)),
            pl.BlockSpec((d, d), lambda i: (0, 0)),
        ],
        out_specs=pl.BlockSpec((blk, 2 * d), lambda i: (i, 0)),
        out_shape=jax.ShapeDtypeStruct((c, 2 * d), jnp.float32),
        compiler_params=pltpu.CompilerParams(
            dimension_semantics=("arbitrary",)),
    )(embed, w0b, b0.reshape(1, d), r0b)


def _sc_gather(table, idx):
    n = idx.shape[0]
    d = table.shape[1]
    nw = 32
    per_w = n // nw
    chunk = 128
    nbuf = 6
    nstream = per_w // chunk
    ngroup = nstream // nbuf
    rem = nstream - ngroup * nbuf
    tail = per_w - nstream * chunk
    idx3 = idx.reshape(nw, 1, per_w)
    mesh = plsc.VectorSubcoreMesh(core_axis_name="c", subcore_axis_name="s")

    @functools.partial(
        pl.kernel,
        mesh=mesh,
        out_type=jax.ShapeDtypeStruct((n, d), jnp.float32),
        scratch_types=[
            pltpu.VMEM((1, per_w), jnp.int32),
            pltpu.VMEM((nbuf, chunk, d), jnp.float32),
            pltpu.SemaphoreType.DMA,
            pltpu.SemaphoreType.DMA,
        ],
    )
    def k(table_hbm, idx_hbm, out_hbm, idx_v, rows_v, gsem, wsem):
        wid = lax.axis_index("s") * 2 + lax.axis_index("c")
        pltpu.sync_copy(idx_hbm.at[wid], idx_v)
        wbase = wid * per_w

        def group(g, carry):
            @pl.when(g > 0)
            def _():
                for b in range(nbuf):
                    pltpu.make_async_copy(table_hbm.at[pl.ds(0, chunk)],
                                          rows_v.at[b], wsem).wait()
            offs = [(g * nbuf + b) * chunk for b in range(nbuf)]
            handles = [
                pltpu.async_copy(
                    table_hbm.at[idx_v.at[0, pl.ds(offs[b], chunk)]],
                    rows_v.at[b], gsem)
                for b in range(nbuf)
            ]
            for h in handles:
                h.wait()
            for b in range(nbuf):
                pltpu.async_copy(rows_v.at[b],
                                 out_hbm.at[pl.ds(wbase + offs[b], chunk)],
                                 wsem)
            return carry

        lax.fori_loop(0, ngroup, group, 0)
        if rem:
            if ngroup > 0:
                for b in range(rem):
                    pltpu.make_async_copy(table_hbm.at[pl.ds(0, chunk)],
                                          rows_v.at[b], wsem).wait()
            roffs = [(ngroup * nbuf + b) * chunk for b in range(rem)]
            rhandles = [
                pltpu.async_copy(
                    table_hbm.at[idx_v.at[0, pl.ds(roffs[b], chunk)]],
                    rows_v.at[b], gsem)
                for b in range(rem)
            ]
            for h in rhandles:
                h.wait()
            for b in range(rem):
                pltpu.async_copy(rows_v.at[b],
                                 out_hbm.at[pl.ds(wbase + roffs[b], chunk)],
                                 wsem)
        for b in range(nbuf if ngroup > 0 else rem):
            pltpu.make_async_copy(table_hbm.at[pl.ds(0, chunk)],
                                  rows_v.at[b], wsem).wait()
        if tail:
            toff = nstream * chunk
            pltpu.async_copy(
                table_hbm.at[idx_v.at[0, pl.ds(toff, tail)]],
                rows_v.at[0, pl.ds(0, tail)], gsem).wait()
            pltpu.sync_copy(rows_v.at[0, pl.ds(0, tail)],
                            out_hbm.at[pl.ds(wbase + toff, tail)])

    return k(table, idx3)


def _tc_mlp(orig_t, react, p):
    ws, bs, rs, w_out, b_out = p
    d, n = orig_t.shape
    blk = 1280
    sub = blk // 128
    grid = n // blk

    bf = jnp.bfloat16
    flat = [ws[0][:d].astype(bf), rs[0][:d].astype(bf)]
    layer_has_r = []
    for i in range(1, len(ws)):
        flat += [ws[i].astype(bf), bs[i].reshape(1, -1)]
        if rs[i] is not None:
            flat.append(rs[i].astype(bf))
            layer_has_r.append(True)
        else:
            layer_has_r.append(False)
    flat += [w_out.reshape(1, -1).astype(bf), b_out.reshape(1, 1)]
    n_w = len(flat)

    def body(*refs):
        orig_ref, react_ref = refs[0], refs[1]
        wv = [r[...] for r in refs[2:2 + n_w]]
        gate_ref = refs[2 + n_w]
        gmax_ref = refs[3 + n_w]
        smax_ref = refs[4 + n_w]

        ot = orig_ref[...].astype(bf)
        it = iter(wv)
        w0a, r0a = next(it), next(it)
        dT = lambda a, w: jax.lax.dot_general(
            a, w, (((0,), (0,)), ((), ())),
            preferred_element_type=jnp.float32)
        z = jnp.maximum(dT(ot, w0a) + react_ref[:, :d], 0.0)
        h = z + dT(ot, r0a) + react_ref[:, d:]
        for has_r in layer_has_r:
            w = next(it)
            b = next(it)
            h16 = h.astype(bf)
            z = jnp.maximum(jnp.dot(h16, w,
                                    preferred_element_type=jnp.float32) + b,
                            0.0)
            res = jnp.dot(h16, next(it),
                          preferred_element_type=jnp.float32) if has_r else h
            h = z + res
        wo = next(it)
        bo = next(it)
        h16 = h.astype(bf)
        rows = []
        for j in range(sub):
            hs = h16[j * 128:(j + 1) * 128, :]
            rows.append(jax.lax.dot_general(
                wo, hs, (((1,), (1,)), ((), ())),
                preferred_element_type=jnp.float32))
        gate = jnp.concatenate(rows, axis=0) + bo[0, 0]
        gate_ref[0] = gate

        bm = jnp.max(gate)
        prev = jnp.where(pl.program_id(0) == 0, -jnp.inf, smax_ref[0])
        m = jnp.maximum(prev, bm)
        smax_ref[0] = m
        gmax_ref[...] = jnp.full((8, 128), m, jnp.float32)

    in_specs = [
        pl.BlockSpec((d, blk), lambda i: (0, i)),
        pl.BlockSpec((blk, 2 * d), lambda i: (i, 0)),
    ]
    for a in flat:
        in_specs.append(pl.BlockSpec(a.shape, lambda i: (0,) * a.ndim))

    gate, gmax = pl.pallas_call(
        body,
        grid=(grid,),
        in_specs=in_specs,
        out_specs=[
            pl.BlockSpec((1, sub, 128), lambda i: (i, 0, 0)),
            pl.BlockSpec((8, 128), lambda i: (0, 0)),
        ],
        out_shape=[
            jax.ShapeDtypeStruct((grid, sub, 128), jnp.float32),
            jax.ShapeDtypeStruct((8, 128), jnp.float32),
        ],
        scratch_shapes=[pltpu.SMEM((1,), jnp.float32)],
        compiler_params=pltpu.CompilerParams(
            dimension_semantics=("arbitrary",)),
    )(orig_t, react, *flat)
    return gate.reshape(n // 128, 128), gmax


def _sc_softmax(gate2, idx2, gmax16, zeros_c):
    r = gate2.shape[0]
    c = zeros_c.shape[0]
    nt = 16
    rb = r // nt
    mesh = plsc.VectorSubcoreMesh(core_axis_name="c", subcore_axis_name="s",
                                  num_cores=1)

    @functools.partial(
        pl.kernel,
        mesh=mesh,
        out_type=jax.ShapeDtypeStruct((r, 128), jnp.float32),
        scratch_types=[
            pltpu.VMEM((rb, 128), jnp.float32),
            pltpu.VMEM((rb, 128), jnp.int32),
            pltpu.VMEM((16,), jnp.float32),
            pltpu.VMEM((128,), jnp.float32),
            pltpu.VMEM_SHARED((c,), jnp.float32),
        ],
    )
    def k(gate_hbm, idx_hbm, gmax_hbm, zc_hbm, out_hbm,
          ebuf, ibuf, gmv, drow, accs):
        t = lax.axis_index("s")
        base = t * rb
        nr = rb

        pltpu.sync_copy(gate_hbm.at[pl.ds(base, rb)], ebuf)
        pltpu.sync_copy(idx_hbm.at[pl.ds(base, rb)], ibuf)
        pltpu.sync_copy(gmax_hbm, gmv)

        @pl.when(t == 0)
        def _():
            pltpu.sync_copy(zc_hbm, accs)

        m = gmv[...]

        def row_exp(ri, carry):
            for li in range(8):
                g = ebuf[ri, pl.ds(li * 16, 16)]
                ebuf[ri, pl.ds(li * 16, 16)] = jnp.exp(g - m)
            return carry

        lax.fori_loop(0, nr, row_exp, 0)
        plsc.subcore_barrier()

        def row_scatter(ri, carry):
            pltpu.sync_copy(ebuf.at[ri], accs.at[ibuf.at[ri]], add=True)
            return carry

        lax.fori_loop(0, nr, row_scatter, 0)
        plsc.subcore_barrier()

        def row_div(ri, carry):
            pltpu.sync_copy(accs.at[ibuf.at[ri]], drow)
            for li in range(8):
                den = drow[pl.ds(li * 16, 16)] + 1e-13
                e = ebuf[ri, pl.ds(li * 16, 16)]
                ebuf[ri, pl.ds(li * 16, 16)] = e / den
            return carry

        lax.fori_loop(0, nr, row_div, 0)

        pltpu.sync_copy(ebuf, out_hbm.at[pl.ds(base, rb)])

    return k(gate2, idx2, gmax16, zeros_c)


def kernel(orig_elem_fea, reaction_elem_idx, reaction_embed, params):
    n = orig_elem_fea.shape[0]
    c = reaction_embed.shape[0]
    idx = reaction_elem_idx.astype(jnp.int32)
    rows = n // 128
    rows_pad = ((rows + 127) // 128) * 128
    pad = rows_pad - rows
    idx2 = jnp.concatenate(
        [idx.reshape(rows, 128),
         jnp.full((pad, 128), c - 1, jnp.int32)], axis=0)
    zeros_c = jnp.zeros((c,), jnp.float32)

    outs = []
    orig_t = orig_elem_fea.T
    for p in params:
        ws, bs, rs, _, _ = p
        d = orig_elem_fea.shape[1]
        table = _tc_project_table(reaction_embed, ws[0][d:], bs[0], rs[0][d:])
        react = _sc_gather(table, idx)
        gate, gmax = _tc_mlp(orig_t, react, p)
        gate2 = jnp.concatenate(
            [gate, jnp.full((pad, 128), -1e30, jnp.float32)], axis=0)
        gmax16 = gmax[0, :16]
        sm = _sc_softmax(gate2, idx2, gmax16, zeros_c)
        outs.append(sm[:rows].reshape(-1))
    if len(outs) == 1:
        return outs[0]
    return jnp.mean(jnp.stack(outs), axis=0)


# speedup vs baseline: 1.3436x; 1.0842x over previous
import functools

import jax
import jax.numpy as jnp
from jax import lax
from jax.experimental import pallas as pl
from jax.experimental.pallas import tpu as pltpu
from jax.experimental.pallas import tpu_sc as plsc


def _tc_project_table(embed, w0b, b0, r0b):
    c, d = embed.shape
    blk = 1000
    grid = c // blk

    def body(e_ref, w_ref, b_ref, r_ref, out_ref):
        e = e_ref[...]
        out_ref[:, :d] = (
            jnp.dot(e, w_ref[...], preferred_element_type=jnp.float32)
            + b_ref[...])
        out_ref[:, d:] = jnp.dot(e, r_ref[...],
                                 preferred_element_type=jnp.float32)

    return pl.pallas_call(
        body,
        grid=(grid,),
        in_specs=[
            pl.BlockSpec((blk, d), lambda i: (i, 0)),
            pl.BlockSpec((d, d), lambda i: (0, 0)),
            pl.BlockSpec((1, d), lambda i: (0, 0)),
            pl.BlockSpec((d, d), lambda i: (0, 0)),
        ],
        out_specs=pl.BlockSpec((blk, 2 * d), lambda i: (i, 0)),
        out_shape=jax.ShapeDtypeStruct((c, 2 * d), jnp.float32),
        compiler_params=pltpu.CompilerParams(
            dimension_semantics=("arbitrary",)),
    )(embed, w0b, b0.reshape(1, d), r0b)


def _sc_gather(table, idx):
    n = idx.shape[0]
    d = table.shape[1]
    nw = 32
    per_w = n // nw
    chunk = 128
    nbuf = 6
    nstream = per_w // chunk
    ngroup = nstream // nbuf
    rem = nstream - ngroup * nbuf
    tail = per_w - nstream * chunk
    idx3 = idx.reshape(nw, 1, per_w)
    mesh = plsc.VectorSubcoreMesh(core_axis_name="c", subcore_axis_name="s")

    @functools.partial(
        pl.kernel,
        mesh=mesh,
        out_type=jax.ShapeDtypeStruct((n, d), jnp.float32),
        scratch_types=[
            pltpu.VMEM((1, per_w), jnp.int32),
            pltpu.VMEM((nbuf, chunk, d), jnp.float32),
            pltpu.SemaphoreType.DMA,
            pltpu.SemaphoreType.DMA,
        ],
    )
    def k(table_hbm, idx_hbm, out_hbm, idx_v, rows_v, gsem, wsem):
        wid = lax.axis_index("s") * 2 + lax.axis_index("c")
        pltpu.sync_copy(idx_hbm.at[wid], idx_v)
        wbase = wid * per_w

        def group(g, carry):
            @pl.when(g > 0)
            def _():
                for b in range(nbuf):
                    pltpu.make_async_copy(table_hbm.at[pl.ds(0, chunk)],
                                          rows_v.at[b], wsem).wait()
            offs = [(g * nbuf + b) * chunk for b in range(nbuf)]
            handles = [
                pltpu.async_copy(
                    table_hbm.at[idx_v.at[0, pl.ds(offs[b], chunk)]],
                    rows_v.at[b], gsem)
                for b in range(nbuf)
            ]
            for h in handles:
                h.wait()
            for b in range(nbuf):
                pltpu.async_copy(rows_v.at[b],
                                 out_hbm.at[pl.ds(wbase + offs[b], chunk)],
                                 wsem)
            return carry

        lax.fori_loop(0, ngroup, group, 0)
        if rem:
            if ngroup > 0:
                for b in range(rem):
                    pltpu.make_async_copy(table_hbm.at[pl.ds(0, chunk)],
                                          rows_v.at[b], wsem).wait()
            roffs = [(ngroup * nbuf + b) * chunk for b in range(rem)]
            rhandles = [
                pltpu.async_copy(
                    table_hbm.at[idx_v.at[0, pl.ds(roffs[b], chunk)]],
                    rows_v.at[b], gsem)
                for b in range(rem)
            ]
            for h in rhandles:
                h.wait()
            for b in range(rem):
                pltpu.async_copy(rows_v.at[b],
                                 out_hbm.at[pl.ds(wbase + roffs[b], chunk)],
                                 wsem)
        for b in range(nbuf if ngroup > 0 else rem):
            pltpu.make_async_copy(table_hbm.at[pl.ds(0, chunk)],
                                  rows_v.at[b], wsem).wait()
        if tail:
            toff = nstream * chunk
            pltpu.async_copy(
                table_hbm.at[idx_v.at[0, pl.ds(toff, tail)]],
                rows_v.at[0, pl.ds(0, tail)], gsem).wait()
            pltpu.sync_copy(rows_v.at[0, pl.ds(0, tail)],
                            out_hbm.at[pl.ds(wbase + toff, tail)])

    return k(table, idx3)


def _tc_mlp(orig_t, react, p, col0=0):
    ws, bs, rs, w_out, b_out = p
    d, _ = orig_t.shape
    n = react.shape[0]
    blk = 1280
    sub = blk // 128
    grid = n // blk
    blk0 = col0 // blk

    bf = jnp.bfloat16
    flat = [ws[0][:d].astype(bf), rs[0][:d].astype(bf)]
    layer_has_r = []
    for i in range(1, len(ws)):
        flat += [ws[i].astype(bf), bs[i].reshape(1, -1)]
        if rs[i] is not None:
            flat.append(rs[i].astype(bf))
            layer_has_r.append(True)
        else:
            layer_has_r.append(False)
    flat += [w_out.reshape(1, -1).astype(bf), b_out.reshape(1, 1)]
    n_w = len(flat)

    def body(*refs):
        orig_ref, react_ref = refs[0], refs[1]
        wv = [r[...] for r in refs[2:2 + n_w]]
        gate_ref = refs[2 + n_w]
        gmax_ref = refs[3 + n_w]
        smax_ref = refs[4 + n_w]

        ot = orig_ref[...].astype(bf)
        it = iter(wv)
        w0a, r0a = next(it), next(it)
        dT = lambda a, w: jax.lax.dot_general(
            a, w, (((0,), (0,)), ((), ())),
            preferred_element_type=jnp.float32)
        z = jnp.maximum(dT(ot, w0a) + react_ref[:, :d], 0.0)
        h = z + dT(ot, r0a) + react_ref[:, d:]
        for has_r in layer_has_r:
            w = next(it)
            b = next(it)
            h16 = h.astype(bf)
            z = jnp.maximum(jnp.dot(h16, w,
                                    preferred_element_type=jnp.float32) + b,
                            0.0)
            res = jnp.dot(h16, next(it),
                          preferred_element_type=jnp.float32) if has_r else h
            h = z + res
        wo = next(it)
        bo = next(it)
        h16 = h.astype(bf)
        rows = []
        for j in range(sub):
            hs = h16[j * 128:(j + 1) * 128, :]
            rows.append(jax.lax.dot_general(
                wo, hs, (((1,), (1,)), ((), ())),
                preferred_element_type=jnp.float32))
        gate = jnp.concatenate(rows, axis=0) + bo[0, 0]
        gate_ref[0] = gate

        bm = jnp.max(gate)
        prev = jnp.where(pl.program_id(0) == 0, -jnp.inf, smax_ref[0])
        m = jnp.maximum(prev, bm)
        smax_ref[0] = m
        gmax_ref[...] = jnp.full((8, 128), m, jnp.float32)

    in_specs = [
        pl.BlockSpec((d, blk), lambda i: (0, i + blk0)),
        pl.BlockSpec((blk, 2 * d), lambda i: (i, 0)),
    ]
    for a in flat:
        in_specs.append(pl.BlockSpec(a.shape, lambda i: (0,) * a.ndim))

    gate, gmax = pl.pallas_call(
        body,
        grid=(grid,),
        in_specs=in_specs,
        out_specs=[
            pl.BlockSpec((1, sub, 128), lambda i: (i, 0, 0)),
            pl.BlockSpec((8, 128), lambda i: (0, 0)),
        ],
        out_shape=[
            jax.ShapeDtypeStruct((grid, sub, 128), jnp.float32),
            jax.ShapeDtypeStruct((8, 128), jnp.float32),
        ],
        scratch_shapes=[pltpu.SMEM((1,), jnp.float32)],
        compiler_params=pltpu.CompilerParams(
            dimension_semantics=("arbitrary",)),
    )(orig_t, react, *flat)
    return gate.reshape(n // 128, 128), gmax


def _sc_softmax(gate2, idx2, gmax16, zeros_c):
    r = gate2.shape[0]
    c = zeros_c.shape[0]
    nt = 16
    rb = r // nt
    mesh = plsc.VectorSubcoreMesh(core_axis_name="c", subcore_axis_name="s",
                                  num_cores=1)

    @functools.partial(
        pl.kernel,
        mesh=mesh,
        out_type=jax.ShapeDtypeStruct((r, 128), jnp.float32),
        scratch_types=[
            pltpu.VMEM((rb, 128), jnp.float32),
            pltpu.VMEM((rb, 128), jnp.int32),
            pltpu.VMEM((16,), jnp.float32),
            pltpu.VMEM((128,), jnp.float32),
            pltpu.VMEM_SHARED((c,), jnp.float32),
        ],
    )
    def k(gate_hbm, idx_hbm, gmax_hbm, zc_hbm, out_hbm,
          ebuf, ibuf, gmv, drow, accs):
        t = lax.axis_index("s")
        base = t * rb
        nr = rb

        pltpu.sync_copy(gate_hbm.at[pl.ds(base, rb)], ebuf)
        pltpu.sync_copy(idx_hbm.at[pl.ds(base, rb)], ibuf)
        pltpu.sync_copy(gmax_hbm, gmv)

        @pl.when(t == 0)
        def _():
            pltpu.sync_copy(zc_hbm, accs)

        m = gmv[...]

        def row_exp(ri, carry):
            for li in range(8):
                g = ebuf[ri, pl.ds(li * 16, 16)]
                ebuf[ri, pl.ds(li * 16, 16)] = jnp.exp(g - m)
            return carry

        lax.fori_loop(0, nr, row_exp, 0)
        plsc.subcore_barrier()

        def row_scatter(ri, carry):
            pltpu.sync_copy(ebuf.at[ri], accs.at[ibuf.at[ri]], add=True)
            return carry

        lax.fori_loop(0, nr, row_scatter, 0)
        plsc.subcore_barrier()

        def row_div(ri, carry):
            pltpu.sync_copy(accs.at[ibuf.at[ri]], drow)
            for li in range(8):
                den = drow[pl.ds(li * 16, 16)] + 1e-13
                e = ebuf[ri, pl.ds(li * 16, 16)]
                ebuf[ri, pl.ds(li * 16, 16)] = e / den
            return carry

        lax.fori_loop(0, nr, row_div, 0)

        pltpu.sync_copy(ebuf, out_hbm.at[pl.ds(base, rb)])

    return k(gate2, idx2, gmax16, zeros_c)


def kernel(orig_elem_fea, reaction_elem_idx, reaction_embed, params):
    n = orig_elem_fea.shape[0]
    c = reaction_embed.shape[0]
    idx = reaction_elem_idx.astype(jnp.int32)
    rows = n // 128
    rows_pad = ((rows + 127) // 128) * 128
    pad = rows_pad - rows
    idx2 = jnp.concatenate(
        [idx.reshape(rows, 128),
         jnp.full((pad, 128), c - 1, jnp.int32)], axis=0)
    zeros_c = jnp.zeros((c,), jnp.float32)

    outs = []
    orig_t = orig_elem_fea.T
    for p in params:
        ws, bs, rs, _, _ = p
        d = orig_elem_fea.shape[1]
        table = _tc_project_table(reaction_embed, ws[0][d:], bs[0], rs[0][d:])
        nphase = 2
        ph = n // nphase
        gates, gmaxes = [], []
        for k in range(nphase):
            react = _sc_gather(table, idx[k * ph:(k + 1) * ph])
            g, gm = _tc_mlp(orig_t, react, p, col0=k * ph)
            gates.append(g)
            gmaxes.append(gm)
        gate = jnp.concatenate(gates, axis=0)
        gmax = gmaxes[0]
        for gm in gmaxes[1:]:
            gmax = jnp.maximum(gmax, gm)
        gate2 = jnp.concatenate(
            [gate, jnp.full((pad, 128), -1e30, jnp.float32)], axis=0)
        gmax16 = gmax[0, :16]
        sm = _sc_softmax(gate2, idx2, gmax16, zeros_c)
        outs.append(sm[:rows].reshape(-1))
    if len(outs) == 1:
        return outs[0]
    return jnp.mean(jnp.stack(outs), axis=0)


# speedup vs baseline: 1.5242x; 1.1345x over previous
import functools

import jax
import jax.numpy as jnp
from jax import lax
from jax.experimental import pallas as pl
from jax.experimental.pallas import tpu as pltpu
from jax.experimental.pallas import tpu_sc as plsc


def _tc_project_table(embed, w0b, b0, r0b):
    c, d = embed.shape
    blk = 1000
    grid = c // blk

    def body(e_ref, w_ref, b_ref, r_ref, out_ref):
        e = e_ref[...]
        out_ref[:, :d] = (
            jnp.dot(e, w_ref[...], preferred_element_type=jnp.float32)
            + b_ref[...])
        out_ref[:, d:] = jnp.dot(e, r_ref[...],
                                 preferred_element_type=jnp.float32)

    return pl.pallas_call(
        body,
        grid=(grid,),
        in_specs=[
            pl.BlockSpec((blk, d), lambda i: (i, 0)),
            pl.BlockSpec((d, d), lambda i: (0, 0)),
            pl.BlockSpec((1, d), lambda i: (0, 0)),
            pl.BlockSpec((d, d), lambda i: (0, 0)),
        ],
        out_specs=pl.BlockSpec((blk, 2 * d), lambda i: (i, 0)),
        out_shape=jax.ShapeDtypeStruct((c, 2 * d), jnp.float32),
        compiler_params=pltpu.CompilerParams(
            dimension_semantics=("arbitrary",)),
    )(embed, w0b, b0.reshape(1, d), r0b)


def _sc_gather(table, idx):
    n = idx.shape[0]
    d = table.shape[1]
    nw = 32
    per_w = n // nw
    chunk = 128
    nbuf = 6
    nstream = per_w // chunk
    ngroup = nstream // nbuf
    rem = nstream - ngroup * nbuf
    tail = per_w - nstream * chunk
    idx3 = idx.reshape(nw, 1, per_w)
    mesh = plsc.VectorSubcoreMesh(core_axis_name="c", subcore_axis_name="s")

    @functools.partial(
        pl.kernel,
        mesh=mesh,
        out_type=jax.ShapeDtypeStruct((n, d), jnp.float32),
        scratch_types=[
            pltpu.VMEM((1, per_w), jnp.int32),
            pltpu.VMEM((nbuf, chunk, d), jnp.float32),
            pltpu.SemaphoreType.DMA,
            pltpu.SemaphoreType.DMA,
        ],
    )
    def k(table_hbm, idx_hbm, out_hbm, idx_v, rows_v, gsem, wsem):
        wid = lax.axis_index("s") * 2 + lax.axis_index("c")
        pltpu.sync_copy(idx_hbm.at[wid], idx_v)
        wbase = wid * per_w

        def group(g, carry):
            @pl.when(g > 0)
            def _():
                for b in range(nbuf):
                    pltpu.make_async_copy(table_hbm.at[pl.ds(0, chunk)],
                                          rows_v.at[b], wsem).wait()
            offs = [(g * nbuf + b) * chunk for b in range(nbuf)]
            handles = [
                pltpu.async_copy(
                    table_hbm.at[idx_v.at[0, pl.ds(offs[b], chunk)]],
                    rows_v.at[b], gsem)
                for b in range(nbuf)
            ]
            for h in handles:
                h.wait()
            for b in range(nbuf):
                pltpu.async_copy(rows_v.at[b],
                                 out_hbm.at[pl.ds(wbase + offs[b], chunk)],
                                 wsem)
            return carry

        lax.fori_loop(0, ngroup, group, 0)
        if rem:
            if ngroup > 0:
                for b in range(rem):
                    pltpu.make_async_copy(table_hbm.at[pl.ds(0, chunk)],
                                          rows_v.at[b], wsem).wait()
            roffs = [(ngroup * nbuf + b) * chunk for b in range(rem)]
            rhandles = [
                pltpu.async_copy(
                    table_hbm.at[idx_v.at[0, pl.ds(roffs[b], chunk)]],
                    rows_v.at[b], gsem)
                for b in range(rem)
            ]
            for h in rhandles:
                h.wait()
            for b in range(rem):
                pltpu.async_copy(rows_v.at[b],
                                 out_hbm.at[pl.ds(wbase + roffs[b], chunk)],
                                 wsem)
        for b in range(nbuf if ngroup > 0 else rem):
            pltpu.make_async_copy(table_hbm.at[pl.ds(0, chunk)],
                                  rows_v.at[b], wsem).wait()
        if tail:
            toff = nstream * chunk
            pltpu.async_copy(
                table_hbm.at[idx_v.at[0, pl.ds(toff, tail)]],
                rows_v.at[0, pl.ds(0, tail)], gsem).wait()
            pltpu.sync_copy(rows_v.at[0, pl.ds(0, tail)],
                            out_hbm.at[pl.ds(wbase + toff, tail)])

    return k(table, idx3)


def _tc_mlp(orig_t, react, p, col0=0):
    ws, bs, rs, w_out, b_out = p
    d, _ = orig_t.shape
    n = react.shape[0]
    blk = 3200
    sub = blk // 128
    grid = n // blk
    blk0 = col0 // blk

    bf = jnp.bfloat16
    flat = [jnp.concatenate([ws[0][:d], rs[0][:d]], axis=1).astype(bf)]
    layer_has_r = []
    for i in range(1, len(ws)):
        if rs[i] is not None:
            flat += [jnp.concatenate([ws[i], rs[i]], axis=1).astype(bf),
                     bs[i].reshape(1, -1)]
            layer_has_r.append(True)
        else:
            flat += [ws[i].astype(bf), bs[i].reshape(1, -1)]
            layer_has_r.append(False)
    flat += [w_out.reshape(1, -1).astype(bf), b_out.reshape(1, 1)]
    n_w = len(flat)

    def body(*refs):
        orig_ref, react_ref = refs[0], refs[1]
        wv = [r[...] for r in refs[2:2 + n_w]]
        gate_ref = refs[2 + n_w]
        gmax_ref = refs[3 + n_w]
        smax_ref = refs[4 + n_w]

        ot = orig_ref[...].astype(bf)
        it = iter(wv)
        wr0 = next(it)
        zr = jax.lax.dot_general(
            ot, wr0, (((0,), (0,)), ((), ())),
            preferred_element_type=jnp.float32)
        z = jnp.maximum(zr[:, :d] + react_ref[:, :d], 0.0)
        h = z + zr[:, d:] + react_ref[:, d:]
        for has_r in layer_has_r:
            w = next(it)
            b = next(it)
            h16 = h.astype(bf)
            if has_r:
                out_d = w.shape[1] // 2
                zr = jnp.dot(h16, w, preferred_element_type=jnp.float32)
                z = jnp.maximum(zr[:, :out_d] + b, 0.0)
                h = z + zr[:, out_d:]
            else:
                z = jnp.maximum(
                    jnp.dot(h16, w, preferred_element_type=jnp.float32) + b,
                    0.0)
                h = z + h
        wo = next(it)
        bo = next(it)
        h16 = h.astype(bf)
        rows = []
        for j in range(sub):
            hs = h16[j * 128:(j + 1) * 128, :]
            rows.append(jax.lax.dot_general(
                wo, hs, (((1,), (1,)), ((), ())),
                preferred_element_type=jnp.float32))
        gate = jnp.concatenate(rows, axis=0) + bo[0, 0]
        gate_ref[0] = gate

        bm = jnp.max(gate)
        prev = jnp.where(pl.program_id(0) == 0, -jnp.inf, smax_ref[0])
        m = jnp.maximum(prev, bm)
        smax_ref[0] = m
        gmax_ref[...] = jnp.full((8, 128), m, jnp.float32)

    in_specs = [
        pl.BlockSpec((d, blk), lambda i: (0, i + blk0)),
        pl.BlockSpec((blk, 2 * d), lambda i: (i, 0)),
    ]
    for a in flat:
        in_specs.append(pl.BlockSpec(a.shape, lambda i: (0,) * a.ndim))

    gate, gmax = pl.pallas_call(
        body,
        grid=(grid,),
        in_specs=in_specs,
        out_specs=[
            pl.BlockSpec((1, sub, 128), lambda i: (i, 0, 0)),
            pl.BlockSpec((8, 128), lambda i: (0, 0)),
        ],
        out_shape=[
            jax.ShapeDtypeStruct((grid, sub, 128), jnp.float32),
            jax.ShapeDtypeStruct((8, 128), jnp.float32),
        ],
        scratch_shapes=[pltpu.SMEM((1,), jnp.float32)],
        compiler_params=pltpu.CompilerParams(
            dimension_semantics=("arbitrary",)),
    )(orig_t, react, *flat)
    return gate.reshape(n // 128, 128), gmax


def _sc_softmax(gate2, idx2, gmax16, zeros_c):
    r = gate2.shape[0]
    c = zeros_c.shape[0]
    nt = 16
    rb = r // nt
    mesh = plsc.VectorSubcoreMesh(core_axis_name="c", subcore_axis_name="s",
                                  num_cores=1)

    @functools.partial(
        pl.kernel,
        mesh=mesh,
        out_type=jax.ShapeDtypeStruct((r, 128), jnp.float32),
        scratch_types=[
            pltpu.VMEM((rb, 128), jnp.float32),
            pltpu.VMEM((rb, 128), jnp.int32),
            pltpu.VMEM((16,), jnp.float32),
            pltpu.VMEM((128,), jnp.float32),
            pltpu.VMEM_SHARED((c,), jnp.float32),
        ],
    )
    def k(gate_hbm, idx_hbm, gmax_hbm, zc_hbm, out_hbm,
          ebuf, ibuf, gmv, drow, accs):
        t = lax.axis_index("s")
        base = t * rb
        nr = rb

        pltpu.sync_copy(gate_hbm.at[pl.ds(base, rb)], ebuf)
        pltpu.sync_copy(idx_hbm.at[pl.ds(base, rb)], ibuf)
        pltpu.sync_copy(gmax_hbm, gmv)

        @pl.when(t == 0)
        def _():
            pltpu.sync_copy(zc_hbm, accs)

        m = gmv[...]

        def row_exp(ri, carry):
            for li in range(8):
                g = ebuf[ri, pl.ds(li * 16, 16)]
                ebuf[ri, pl.ds(li * 16, 16)] = jnp.exp(g - m)
            return carry

        lax.fori_loop(0, nr, row_exp, 0)
        plsc.subcore_barrier()

        def row_scatter(ri, carry):
            pltpu.sync_copy(ebuf.at[ri], accs.at[ibuf.at[ri]], add=True)
            return carry

        lax.fori_loop(0, nr, row_scatter, 0)
        plsc.subcore_barrier()

        def row_div(ri, carry):
            pltpu.sync_copy(accs.at[ibuf.at[ri]], drow)
            for li in range(8):
                den = drow[pl.ds(li * 16, 16)] + 1e-13
                e = ebuf[ri, pl.ds(li * 16, 16)]
                ebuf[ri, pl.ds(li * 16, 16)] = e / den
            return carry

        lax.fori_loop(0, nr, row_div, 0)

        pltpu.sync_copy(ebuf, out_hbm.at[pl.ds(base, rb)])

    return k(gate2, idx2, gmax16, zeros_c)


def kernel(orig_elem_fea, reaction_elem_idx, reaction_embed, params):
    n = orig_elem_fea.shape[0]
    c = reaction_embed.shape[0]
    idx = reaction_elem_idx.astype(jnp.int32)
    rows = n // 128
    rows_pad = ((rows + 127) // 128) * 128
    pad = rows_pad - rows
    idx2 = jnp.concatenate(
        [idx.reshape(rows, 128),
         jnp.full((pad, 128), c - 1, jnp.int32)], axis=0)
    zeros_c = jnp.zeros((c,), jnp.float32)

    outs = []
    orig_t = orig_elem_fea.T
    for p in params:
        ws, bs, rs, _, _ = p
        d = orig_elem_fea.shape[1]
        table = _tc_project_table(reaction_embed, ws[0][d:], bs[0], rs[0][d:])
        nphase = 2
        ph = n // nphase
        gates, gmaxes = [], []
        for k in range(nphase):
            react = _sc_gather(table, idx[k * ph:(k + 1) * ph])
            g, gm = _tc_mlp(orig_t, react, p, col0=k * ph)
            gates.append(g)
            gmaxes.append(gm)
        gate = jnp.concatenate(gates, axis=0)
        gmax = gmaxes[0]
        for gm in gmaxes[1:]:
            gmax = jnp.maximum(gmax, gm)
        gate2 = jnp.concatenate(
            [gate, jnp.full((pad, 128), -1e30, jnp.float32)], axis=0)
        gmax16 = gmax[0, :16]
        sm = _sc_softmax(gate2, idx2, gmax16, zeros_c)
        outs.append(sm[:rows].reshape(-1))
    if len(outs) == 1:
        return outs[0]
    return jnp.mean(jnp.stack(outs), axis=0)
